# baseline (speedup 1.0000x reference)
import functools

import jax
import jax.numpy as jnp
from jax import lax
from jax.experimental import pallas as pl
from jax.experimental.pallas import tpu as pltpu
from jax.experimental.pallas import tpu_sc as plsc

B = 16384
IMG_IN = 512
DU = 64
DI = 32
NC = 2
NS = 16
NW = NC * NS
CW = 512
CSH = 9
SLOTG = 128
JUNK = B
OUTR = B + 16

BB = 512
GRID = B // BB

_MESH = plsc.VectorSubcoreMesh(core_axis_name="c", subcore_axis_name="s")

_I16 = lambda: lax.iota(jnp.int32, 16)


def _make_filter_kernel(V, NF, tail_tiles):
    KF = V // CW
    TAILW = V - KF * CW
    TMAX = (KF - 1) // NW + 1
    NJ = NF // 16
    TAIL_OWNER = KF % NW

    @functools.partial(
        pl.kernel,
        out_type=(
            jax.ShapeDtypeStruct((OUTR, 128), jnp.float32),
            jax.ShapeDtypeStruct((OUTR,), jnp.float32),
        ),
        mesh=_MESH,
        compiler_params=pltpu.CompilerParams(needs_layout_passes=False, disable_bounds_checks=True),
        scratch_types=(
            pltpu.VMEM((4096,), jnp.int32),
            pltpu.VMEM((B + 16,), jnp.int32),
            pltpu.VMEM((B + 16,), jnp.int32),
            pltpu.VMEM((NF, CW), jnp.float32),
            pltpu.VMEM((NF, CW), jnp.float32),
            pltpu.VMEM((1, CW), jnp.float32),
            pltpu.VMEM((1, CW), jnp.float32),
            pltpu.VMEM((NF, 128 * tail_tiles), jnp.float32),
            pltpu.VMEM((1, 128 * tail_tiles), jnp.float32),
            pltpu.VMEM((16, 128), jnp.float32),
            pltpu.VMEM((16,), jnp.float32),
            pltpu.SemaphoreType.DMA,
            pltpu.SemaphoreType.DMA,
        ),
    )
    def k(idx_hbm, fT_hbm, bT_hbm, rows_out, bias_out,
          sbuf, hu_v, hp_v, cA, cB, bA, bB, tC, tB_,
          stage, bstage, semc, sems):
        wid = lax.axis_index("s") * NC + lax.axis_index("c")

        def scan_q(q, off):
            def scan_g(g, off):
                for u in range(4):
                    iv = sbuf[pl.ds(g * 64 + u * 16, 16)]
                    ck = lax.shift_right_logical(iv, CSH)
                    msk = (ck & (NW - 1)) == wid
                    pc = plsc.all_reduce_population_count(msk)
                    cnt = jnp.max(pc)
                    plsc.store_compressed(hu_v.at[pl.ds(off, 16)], iv, mask=msk)
                    plsc.store_compressed(
                        hp_v.at[pl.ds(off, 16)],
                        _I16() + (q * 4096 + g * 64 + u * 16), mask=msk)
                    off = off + cnt
                return off
            pltpu.sync_copy(idx_hbm.at[pl.ds(q * 4096, 4096)], sbuf)
            return lax.fori_loop(0, 64, scan_g, off)

        total = lax.fori_loop(0, 4, scan_q, 0)
        hu_v[pl.ds(total, 16)] = jnp.full((16,), 0x7FFFFFFF, jnp.int32)
        hp_v[pl.ds(total, 16)] = jnp.full((16,), JUNK, jnp.int32)
        ng = lax.shift_right_logical(total + 15, 4)

        def process(cbuf, bbuf, k_id, cs):
            def group_body(g):
                hu16 = hu_v[pl.ds(g * 16, 16)]
                hp16 = hp_v[pl.ds(g * 16, 16)]
                cm = lax.shift_right_logical(hu16, CSH) == k_id

                @pl.when(jnp.any(cm))
                def _():
                    po = jnp.where(cm, hp16, JUNK)
                    cl = jnp.where(cm, hu16 - cs, 0)
                    bstage[pl.ds(0, 16)] = plsc.load_gather(
                        bbuf, [jnp.zeros((16,), jnp.int32), cl])
                    for i in range(16):
                        ci = cl[jnp.full((16,), 0, jnp.int32) + i]
                        for j in range(NJ):
                            stage[i, pl.ds(16 * j, 16)] = plsc.load_gather(
                                cbuf, [_I16() + 16 * j, ci])
                    cp1 = pltpu.async_copy(stage, rows_out.at[po], sems)
                    cp2 = pltpu.async_copy(bstage, bias_out.at[po], sems)
                    cp1.wait()
                    cp2.wait()

            def fast_g(g, _):
                @pl.when(g * 16 < total)
                def _():
                    group_body(g)
                return 0

            lax.fori_loop(0, SLOTG, fast_g, 0)

            @pl.when(total > SLOTG * 16)
            def _():
                def slow_g(g, _):
                    group_body(g)
                    return 0
                lax.fori_loop(SLOTG, ng, slow_g, 0)

        def issue(k_id, cbuf, bbuf):
            cs = k_id * CW
            pltpu.async_copy(fT_hbm.at[:, pl.ds(cs, CW)], cbuf, semc)
            pltpu.async_copy(bT_hbm.at[:, pl.ds(cs, CW)], bbuf, semc)

        def wait_chunk(cbuf, bbuf):
            pltpu.make_async_copy(fT_hbm.at[:, pl.ds(0, CW)], cbuf, semc).wait()
            pltpu.make_async_copy(bT_hbm.at[:, pl.ds(0, CW)], bbuf, semc).wait()

        @pl.when(wid < KF)
        def _():
            issue(wid, cA, bA)

        def chunk_t(t, carry):
            k_id = wid + NW * t
            k_next = k_id + NW

            def step(cur, bcur, nxt, bnxt):
                @pl.when(k_next < KF)
                def _():
                    issue(k_next, nxt, bnxt)

                @pl.when(k_id < KF)
                def _():
                    wait_chunk(cur, bcur)
                    process(cur, bcur, k_id, k_id * CW)

            @pl.when((t & 1) == 0)
            def _():
                step(cA, bA, cB, bB)

            @pl.when((t & 1) == 1)
            def _():
                step(cB, bB, cA, bA)
            return carry

        lax.fori_loop(0, TMAX, chunk_t, 0)

        if TAILW:
            @pl.when(wid == TAIL_OWNER)
            def _():
                ts0 = KF * CW + wid * 0
                for t in range(tail_tiles):
                    pltpu.async_copy(
                        fT_hbm.at[:, pl.ds(ts0 + 128 * t, 128)],
                        tC.at[:, pl.ds(128 * t, 128)], semc)
                    pltpu.async_copy(
                        bT_hbm.at[:, pl.ds(ts0 + 128 * t, 128)],
                        tB_.at[:, pl.ds(128 * t, 128)], semc)
                for t in range(tail_tiles):
                    pltpu.make_async_copy(
                        fT_hbm.at[:, pl.ds(0, 128)],
                        tC.at[:, pl.ds(128 * t, 128)], semc).wait()
                    pltpu.make_async_copy(
                        bT_hbm.at[:, pl.ds(0, 128)],
                        tB_.at[:, pl.ds(128 * t, 128)], semc).wait()
                process(tC, tB_, KF, KF * CW)

    return k


_filter_user = _make_filter_kernel(1000000, DU, 1)
_filter_item = _make_filter_kernel(100000, DI, 2)


def _tc_body(img_ref, w_ref, b_ref, u_ref, it_ref, ub_ref, ib_ref, o_ref):
    img = jnp.dot(img_ref[...], w_ref[...], preferred_element_type=jnp.float32)
    img = img + b_ref[...]
    u = u_ref[...]
    t = u[:, :DI] * img + u[:, DI:DU] * it_ref[:, :DI]
    o_ref[...] = jnp.sum(t, axis=1) + ub_ref[...] + ib_ref[...]


def kernel(image, user, item, user_factors, item_factors, user_biases,
           item_biases, W_img, b_img):
    user = user.astype(jnp.int32)
    item = item.astype(jnp.int32)
    urows, ub = _filter_user(user, user_factors.T, user_biases.T)
    irows, ib = _filter_item(item, item_factors.T, item_biases.T)
    out = pl.pallas_call(
        _tc_body,
        grid=(GRID,),
        in_specs=[
            pl.BlockSpec((BB, IMG_IN), lambda i: (i, 0)),
            pl.BlockSpec((IMG_IN, DI), lambda i: (0, 0)),
            pl.BlockSpec((1, DI), lambda i: (0, 0)),
            pl.BlockSpec((BB, 128), lambda i: (i, 0)),
            pl.BlockSpec((BB, 128), lambda i: (i, 0)),
            pl.BlockSpec((BB,), lambda i: (i,)),
            pl.BlockSpec((BB,), lambda i: (i,)),
        ],
        out_specs=pl.BlockSpec((BB,), lambda i: (i,)),
        out_shape=jax.ShapeDtypeStruct((B,), jnp.float32),
    )(image, W_img, b_img.reshape(1, DI), urows, irows, ub, ib)
    return out


# speedup vs baseline: 105.1330x; 105.1330x over previous
import functools

import jax
import jax.numpy as jnp
from jax import lax
from jax.experimental import pallas as pl
from jax.experimental.pallas import tpu as pltpu
from jax.experimental.pallas import tpu_sc as plsc

B = 16384
IMG_IN = 512
DU = 64
DI = 32
NC = 2
NS = 16
NW = NC * NS
BPW = B // NW
RING = 4

BB = 512
GRID = B // BB

_MESH = plsc.VectorSubcoreMesh(core_axis_name="c", subcore_axis_name="s")

_I16 = lambda: lax.iota(jnp.int32, 16)


@functools.partial(
    pl.kernel,
    out_type=(
        jax.ShapeDtypeStruct((B, DU), jnp.float32),
        jax.ShapeDtypeStruct((B,), jnp.float32),
    ),
    mesh=_MESH,
    compiler_params=pltpu.CompilerParams(
        needs_layout_passes=False, disable_bounds_checks=True),
    scratch_types=(
        pltpu.VMEM((BPW,), jnp.int32),
        pltpu.VMEM((BPW,), jnp.int32),
        pltpu.VMEM((BPW, DU), jnp.float32),
        pltpu.VMEM((BPW,), jnp.float32),
        pltpu.VMEM((RING, DU, 128), jnp.float32),
        pltpu.VMEM((RING, 1, 128), jnp.float32),
        pltpu.VMEM((RING, 1, 128), jnp.float32),
        pltpu.SemaphoreType.DMA,
    ),
)
def _sc_user(uidx_hbm, iidx_hbm, ufT_hbm, ubT_hbm, ibT_hbm,
             urows_out, bias_out,
             uidx_v, iidx_v, urows_v, bsum_v,
             uring, ubring, ibring, sem):
    wid = lax.axis_index("s") * NC + lax.axis_index("c")
    base = wid * BPW
    pltpu.sync_copy(uidx_hbm.at[pl.ds(base, BPW)], uidx_v)
    pltpu.sync_copy(iidx_hbm.at[pl.ds(base, BPW)], iidx_v)

    def group(g, carry):
        uiv = uidx_v[pl.ds(g * 16, 16)]
        iiv = iidx_v[pl.ds(g * 16, 16)]
        uts = lax.shift_left(lax.shift_right_logical(uiv, 7), 7)
        ucc = uiv & 127
        its = lax.shift_left(lax.shift_right_logical(iiv, 7), 7)
        icc = iiv & 127
        ust = [pl.multiple_of(jnp.max(jnp.where(_I16() == i, uts, 0)), 128)
               for i in range(16)]
        ist = [pl.multiple_of(jnp.max(jnp.where(_I16() == i, its, 0)), 128)
               for i in range(16)]

        def fire(i):
            r = i % RING
            return [
                pltpu.async_copy(ufT_hbm.at[:, pl.ds(ust[i], 128)],
                                 uring.at[r], sem),
                pltpu.async_copy(ubT_hbm.at[:, pl.ds(ust[i], 128)],
                                 ubring.at[r], sem),
                pltpu.async_copy(ibT_hbm.at[:, pl.ds(ist[i], 128)],
                                 ibring.at[r], sem),
            ]

        cps = {}
        for i in range(RING):
            cps[i] = fire(i)

        bias_row = jnp.zeros((16,), jnp.float32)
        for i in range(16):
            for cp in cps[i]:
                cp.wait()
            r = i % RING
            ucv = jnp.full((16,), 0, jnp.int32) + jnp.max(
                jnp.where(_I16() == i, ucc, 0))
            icv = jnp.full((16,), 0, jnp.int32) + jnp.max(
                jnp.where(_I16() == i, icc, 0))
            for j in range(DU // 16):
                urows_v[g * 16 + i, pl.ds(16 * j, 16)] = plsc.load_gather(
                    uring.at[r], [_I16() + 16 * j, ucv])
            zz = jnp.zeros((16,), jnp.int32)
            ubv = plsc.load_gather(ubring.at[r], [zz, ucv])
            ibv = plsc.load_gather(ibring.at[r], [zz, icv])
            bias_row = jnp.where(_I16() == i, ubv + ibv, bias_row)
            if i + RING < 16:
                cps[i + RING] = fire(i + RING)
        bsum_v[pl.ds(g * 16, 16)] = bias_row
        return carry

    lax.fori_loop(0, BPW // 16, group, 0)
    pltpu.sync_copy(urows_v, urows_out.at[pl.ds(base, BPW)])
    pltpu.sync_copy(bsum_v, bias_out.at[pl.ds(base, BPW)])


@functools.partial(
    pl.kernel,
    out_type=jax.ShapeDtypeStruct((B, DI), jnp.float32),
    mesh=_MESH,
    compiler_params=pltpu.CompilerParams(
        needs_layout_passes=False, disable_bounds_checks=True),
    scratch_types=(
        pltpu.VMEM((BPW,), jnp.int32),
        pltpu.VMEM((BPW, DI), jnp.float32),
        pltpu.VMEM((RING, DI, 128), jnp.float32),
        pltpu.SemaphoreType.DMA,
    ),
)
def _sc_item(iidx_hbm, ifT_hbm, irows_out, iidx_v, irows_v, iring, sem):
    wid = lax.axis_index("s") * NC + lax.axis_index("c")
    base = wid * BPW
    pltpu.sync_copy(iidx_hbm.at[pl.ds(base, BPW)], iidx_v)

    def group(g, carry):
        iiv = iidx_v[pl.ds(g * 16, 16)]
        its = lax.shift_left(lax.shift_right_logical(iiv, 7), 7)
        icc = iiv & 127
        ist = [pl.multiple_of(jnp.max(jnp.where(_I16() == i, its, 0)), 128)
               for i in range(16)]

        def fire(i):
            return [pltpu.async_copy(ifT_hbm.at[:, pl.ds(ist[i], 128)],
                                     iring.at[i % RING], sem)]

        cps = {}
        for i in range(RING):
            cps[i] = fire(i)

        for i in range(16):
            for cp in cps[i]:
                cp.wait()
            r = i % RING
            icv = jnp.full((16,), 0, jnp.int32) + jnp.max(
                jnp.where(_I16() == i, icc, 0))
            for j in range(DI // 16):
                irows_v[g * 16 + i, pl.ds(16 * j, 16)] = plsc.load_gather(
                    iring.at[r], [_I16() + 16 * j, icv])
            if i + RING < 16:
                cps[i + RING] = fire(i + RING)
        return carry

    lax.fori_loop(0, BPW // 16, group, 0)
    pltpu.sync_copy(irows_v, irows_out.at[pl.ds(base, BPW)])


def _tc_body(img_ref, w_ref, b_ref, u_ref, it_ref, bs_ref, o_ref):
    img = jnp.dot(img_ref[...], w_ref[...], preferred_element_type=jnp.float32)
    img = img + b_ref[...]
    u = u_ref[...]
    t = u[:, :DI] * img + u[:, DI:] * it_ref[...]
    o_ref[...] = jnp.sum(t, axis=1) + bs_ref[...]


def kernel(image, user, item, user_factors, item_factors, user_biases,
           item_biases, W_img, b_img):
    user = user.astype(jnp.int32)
    item = item.astype(jnp.int32)
    urows, bsum = _sc_user(user, item, user_factors.T,
                           user_biases.T, item_biases.T)
    irows = _sc_item(item, item_factors.T)
    out = pl.pallas_call(
        _tc_body,
        grid=(GRID,),
        in_specs=[
            pl.BlockSpec((BB, IMG_IN), lambda i: (i, 0)),
            pl.BlockSpec((IMG_IN, DI), lambda i: (0, 0)),
            pl.BlockSpec((1, DI), lambda i: (0, 0)),
            pl.BlockSpec((BB, DU), lambda i: (i, 0)),
            pl.BlockSpec((BB, DI), lambda i: (i, 0)),
            pl.BlockSpec((BB,), lambda i: (i,)),
        ],
        out_specs=pl.BlockSpec((BB,), lambda i: (i,)),
        out_shape=jax.ShapeDtypeStruct((B,), jnp.float32),
    )(image, W_img, b_img.reshape(1, DI), urows, irows, bsum)
    return out


# speedup vs baseline: 113.0156x; 1.0750x over previous
import functools

import jax
import jax.numpy as jnp
from jax import lax
from jax.experimental import pallas as pl
from jax.experimental.pallas import tpu as pltpu
from jax.experimental.pallas import tpu_sc as plsc

B = 16384
IMG_IN = 512
DU = 64
DI = 32
NC = 2
NS = 16
NW = NC * NS
BPW = B // NW
RING = 6

BB = 512
GRID = B // BB

_MESH = plsc.VectorSubcoreMesh(core_axis_name="c", subcore_axis_name="s")

_I16 = lambda: lax.iota(jnp.int32, 16)


@functools.partial(
    pl.kernel,
    out_type=(
        jax.ShapeDtypeStruct((B, DU), jnp.float32),
        jax.ShapeDtypeStruct((B,), jnp.float32),
    ),
    mesh=_MESH,
    compiler_params=pltpu.CompilerParams(
        needs_layout_passes=False, disable_bounds_checks=True),
    scratch_types=(
        pltpu.VMEM((BPW,), jnp.int32),
        pltpu.VMEM((BPW,), jnp.int32),
        pltpu.VMEM((BPW, DU), jnp.float32),
        pltpu.VMEM((BPW,), jnp.float32),
        pltpu.VMEM((RING, DU, 128), jnp.float32),
        pltpu.VMEM((RING, 1, 128), jnp.float32),
        pltpu.VMEM((RING, 1, 128), jnp.float32),
        pltpu.SemaphoreType.DMA,
    ),
)
def _sc_user(uidx_hbm, iidx_hbm, ufT_hbm, ubT_hbm, ibT_hbm,
             urows_out, bias_out,
             uidx_v, iidx_v, urows_v, bsum_v,
             uring, ubring, ibring, sem):
    wid = lax.axis_index("s") * NC + lax.axis_index("c")
    base = wid * BPW
    pltpu.sync_copy(uidx_hbm.at[pl.ds(base, BPW)], uidx_v)
    pltpu.sync_copy(iidx_hbm.at[pl.ds(base, BPW)], iidx_v)

    def group(g, carry):
        uiv = uidx_v[pl.ds(g * 16, 16)]
        iiv = iidx_v[pl.ds(g * 16, 16)]
        uts = lax.shift_left(lax.shift_right_logical(uiv, 7), 7)
        ucc = uiv & 127
        its = lax.shift_left(lax.shift_right_logical(iiv, 7), 7)
        icc = iiv & 127
        ust = [pl.multiple_of(jnp.max(jnp.where(_I16() == i, uts, 0)), 128)
               for i in range(16)]
        ist = [pl.multiple_of(jnp.max(jnp.where(_I16() == i, its, 0)), 128)
               for i in range(16)]

        def fire(i):
            r = i % RING
            return [
                pltpu.async_copy(ufT_hbm.at[:, pl.ds(ust[i], 128)],
                                 uring.at[r], sem),
                pltpu.async_copy(ubT_hbm.at[:, pl.ds(ust[i], 128)],
                                 ubring.at[r], sem),
                pltpu.async_copy(ibT_hbm.at[:, pl.ds(ist[i], 128)],
                                 ibring.at[r], sem),
            ]

        cps = {}
        for i in range(RING):
            cps[i] = fire(i)

        bias_row = jnp.zeros((16,), jnp.float32)
        for i in range(16):
            for cp in cps[i]:
                cp.wait()
            r = i % RING
            ucv = jnp.full((16,), 0, jnp.int32) + jnp.max(
                jnp.where(_I16() == i, ucc, 0))
            icv = jnp.full((16,), 0, jnp.int32) + jnp.max(
                jnp.where(_I16() == i, icc, 0))
            for j in range(DU // 16):
                urows_v[g * 16 + i, pl.ds(16 * j, 16)] = plsc.load_gather(
                    uring.at[r], [_I16() + 16 * j, ucv])
            zz = jnp.zeros((16,), jnp.int32)
            ubv = plsc.load_gather(ubring.at[r], [zz, ucv])
            ibv = plsc.load_gather(ibring.at[r], [zz, icv])
            bias_row = jnp.where(_I16() == i, ubv + ibv, bias_row)
            if i + RING < 16:
                cps[i + RING] = fire(i + RING)
        bsum_v[pl.ds(g * 16, 16)] = bias_row
        return carry

    lax.fori_loop(0, BPW // 16, group, 0)
    pltpu.sync_copy(urows_v, urows_out.at[pl.ds(base, BPW)])
    pltpu.sync_copy(bsum_v, bias_out.at[pl.ds(base, BPW)])


@functools.partial(
    pl.kernel,
    out_type=jax.ShapeDtypeStruct((B, DI), jnp.float32),
    mesh=_MESH,
    compiler_params=pltpu.CompilerParams(
        needs_layout_passes=False, disable_bounds_checks=True),
    scratch_types=(
        pltpu.VMEM((BPW,), jnp.int32),
        pltpu.VMEM((BPW, DI), jnp.float32),
        pltpu.VMEM((RING, DI, 128), jnp.float32),
        pltpu.SemaphoreType.DMA,
    ),
)
def _sc_item(iidx_hbm, ifT_hbm, irows_out, iidx_v, irows_v, iring, sem):
    wid = lax.axis_index("s") * NC + lax.axis_index("c")
    base = wid * BPW
    pltpu.sync_copy(iidx_hbm.at[pl.ds(base, BPW)], iidx_v)

    def group(g, carry):
        iiv = iidx_v[pl.ds(g * 16, 16)]
        its = lax.shift_left(lax.shift_right_logical(iiv, 7), 7)
        icc = iiv & 127
        ist = [pl.multiple_of(jnp.max(jnp.where(_I16() == i, its, 0)), 128)
               for i in range(16)]

        def fire(i):
            return [pltpu.async_copy(ifT_hbm.at[:, pl.ds(ist[i], 128)],
                                     iring.at[i % RING], sem)]

        cps = {}
        for i in range(RING):
            cps[i] = fire(i)

        for i in range(16):
            for cp in cps[i]:
                cp.wait()
            r = i % RING
            icv = jnp.full((16,), 0, jnp.int32) + jnp.max(
                jnp.where(_I16() == i, icc, 0))
            for j in range(DI // 16):
                irows_v[g * 16 + i, pl.ds(16 * j, 16)] = plsc.load_gather(
                    iring.at[r], [_I16() + 16 * j, icv])
            if i + RING < 16:
                cps[i + RING] = fire(i + RING)
        return carry

    lax.fori_loop(0, BPW // 16, group, 0)
    pltpu.sync_copy(irows_v, irows_out.at[pl.ds(base, BPW)])


def _tc_body(img_ref, w_ref, b_ref, u_ref, it_ref, bs_ref, o_ref):
    img = jnp.dot(img_ref[...], w_ref[...], preferred_element_type=jnp.float32)
    img = img + b_ref[...]
    u = u_ref[...]
    t = u[:, :DI] * img + u[:, DI:] * it_ref[...]
    o_ref[...] = jnp.sum(t, axis=1) + bs_ref[...]


def kernel(image, user, item, user_factors, item_factors, user_biases,
           item_biases, W_img, b_img):
    user = user.astype(jnp.int32)
    item = item.astype(jnp.int32)
    urows, bsum = _sc_user(user, item, user_factors.T,
                           user_biases.T, item_biases.T)
    irows = _sc_item(item, item_factors.T)
    out = pl.pallas_call(
        _tc_body,
        grid=(GRID,),
        in_specs=[
            pl.BlockSpec((BB, IMG_IN), lambda i: (i, 0)),
            pl.BlockSpec((IMG_IN, DI), lambda i: (0, 0)),
            pl.BlockSpec((1, DI), lambda i: (0, 0)),
            pl.BlockSpec((BB, DU), lambda i: (i, 0)),
            pl.BlockSpec((BB, DI), lambda i: (i, 0)),
            pl.BlockSpec((BB,), lambda i: (i,)),
        ],
        out_specs=pl.BlockSpec((BB,), lambda i: (i,)),
        out_shape=jax.ShapeDtypeStruct((B,), jnp.float32),
    )(image, W_img, b_img.reshape(1, DI), urows, irows, bsum)
    return out
